# Initial kernel scaffold; baseline (speedup 1.0000x reference)
import jax
import jax.numpy as jnp
from jax.experimental import pallas as pl


def kernel(q_pts, s_pts, s_feats, neighb_inds, da_scale, W1, b1, W2, weights, kernel_points):
    raise NotImplementedError("write your pallas kernel here")


# speedup vs baseline: 1.1963x; 1.1963x over previous
import functools

import jax
import jax.numpy as jnp
from jax import lax
from jax.experimental import pallas as pl
from jax.experimental.pallas import tpu as pltpu
from jax.experimental.pallas import tpu_sc as plsc

_SIGMA = 2.0
_NEG_SLOPE = 0.1


def _mlp_body(x_ref, w1_ref, b1_ref, w2_ref, out_ref):
    h = jnp.dot(x_ref[...], w1_ref[...], preferred_element_type=jnp.float32)
    h = h + b1_ref[...]
    h = jnp.where(h >= 0.0, h, _NEG_SLOPE * h)
    z = jnp.dot(h, w2_ref[...], preferred_element_type=jnp.float32)
    out_ref[...] = jax.nn.sigmoid(z)


def _mlp(s_feats, W1, b1, W2):
    m, c = s_feats.shape
    cout = W2.shape[1]
    bm = 400
    assert m % bm == 0
    return pl.pallas_call(
        _mlp_body,
        grid=(m // bm,),
        in_specs=[
            pl.BlockSpec((bm, c), lambda i: (i, 0)),
            pl.BlockSpec((c, c), lambda i: (0, 0)),
            pl.BlockSpec((1, c), lambda i: (0, 0)),
            pl.BlockSpec((c, cout), lambda i: (0, 0)),
        ],
        out_specs=pl.BlockSpec((bm, cout), lambda i: (i, 0)),
        out_shape=jax.ShapeDtypeStruct((m, cout), jnp.float32),
    )(s_feats, W1, b1.reshape(1, c), W2)


def _sc_aggregate(q4, s_pts, s_feats, neighb_inds, mod, weights, kernel_points):
    m, c = s_feats.shape
    h = neighb_inds.shape[1]
    k = weights.shape[0]
    n_vreg = c // 16

    info = plsc.get_sparse_core_info()
    nw = info.num_cores * info.num_subcores
    nm = 320
    assert nw * nm >= m

    mesh = plsc.VectorSubcoreMesh(core_axis_name="c", subcore_axis_name="s")

    @functools.partial(
        pl.kernel,
        mesh=mesh,
        compiler_params=pltpu.CompilerParams(
            needs_layout_passes=False, use_tc_tiling_on_sc=False),
        out_type=jax.ShapeDtypeStruct((m, c), jnp.float32),
        scratch_types=[
            pltpu.VMEM((m, 3), jnp.float32),
            pltpu.VMEM((nm, h), jnp.int32),
            pltpu.VMEM((nm, 4), jnp.float32),
            pltpu.VMEM((k, c), jnp.float32),
            pltpu.VMEM((16, 3), jnp.float32),
            pltpu.VMEM((mod.shape[1],), jnp.float32),
            pltpu.VMEM((h, c), jnp.float32),
            pltpu.VMEM((40, c), jnp.float32),
            pltpu.SemaphoreType.DMA,
        ],
    )
    def sc_kernel(q4_hbm, spts_hbm, sfeats_hbm, inds_hbm, mod_hbm, w_hbm,
                  kp_hbm, out_hbm, spts_v, inds_v, q4_v, w_v, kp_v, mod_v,
                  feats_v, outbuf_v, sem):
        wid = lax.axis_index("s") * info.num_cores + lax.axis_index("c")
        base = jnp.minimum(wid * nm, m - nm)

        pltpu.sync_copy(spts_hbm, spts_v)
        pltpu.sync_copy(inds_hbm.at[pl.ds(base, nm)], inds_v)
        pltpu.sync_copy(q4_hbm.at[pl.ds(base, nm)], q4_v)
        pltpu.sync_copy(w_hbm, w_v)
        pltpu.sync_copy(kp_hbm, kp_v.at[pl.ds(0, k)])

        lane = lax.iota(jnp.int32, 16)
        grp = jnp.right_shift(lane, 3)
        zeros16 = jnp.zeros((16,), jnp.int32)

        kpx = plsc.load_gather(kp_v, [lane, zeros16])
        kpy = plsc.load_gather(kp_v, [lane, zeros16 + 1])
        kpz = plsc.load_gather(kp_v, [lane, zeros16 + 2])

        def chunk_body(cc, _):
            def m_body(g2, _inner):
                g = cc * 40 + g2
                pltpu.sync_copy(mod_hbm.at[base + g], mod_v)
                feat_dma = pltpu.async_copy(
                    sfeats_hbm.at[inds_v.at[g]], feats_v, sem)

                qv = plsc.load_gather(
                    q4_v, [jnp.full((16,), g, jnp.int32), lane & 3])
                qx, qy, qz, da = qv[0], qv[1], qv[2], qv[3]

                bks = []
                infls = []
                for half in range(h // 16):
                    nidx = inds_v[g, pl.ds(16 * half, 16)]
                    nx = plsc.load_gather(spts_v, [nidx, zeros16]) - qx
                    ny = plsc.load_gather(spts_v, [nidx, zeros16 + 1]) - qy
                    nz = plsc.load_gather(spts_v, [nidx, zeros16 + 2]) - qz
                    bmin = jnp.full((16,), 1e30, jnp.float32)
                    bk = jnp.zeros((16,), jnp.int32)
                    for kk in range(k):
                        dx = nx - da * kpx[kk]
                        dy = ny - da * kpy[kk]
                        dz = nz - da * kpz[kk]
                        sq = dx * dx + dy * dy + dz * dz
                        better = sq < bmin
                        bk = jnp.where(better, kk, bk)
                        bmin = jnp.where(better, sq, bmin)
                    yi = jnp.int32(0x1FBD1DF5) + jnp.right_shift(
                        plsc.bitcast(bmin, jnp.int32), 1)
                    y = plsc.bitcast(yi, jnp.float32)
                    for _ in range(3):
                        y = 0.5 * (y + bmin / y)
                    infl = jnp.maximum(1.0 - y * (1.0 / _SIGMA), 0.0)
                    bks.append(bk)
                    infls.append(infl)

                feat_dma.wait()

                acc = [jnp.zeros((16,), jnp.float32) for _ in range(n_vreg)]
                for half in range(h // 16):
                    for i in range(16):
                        hh = 16 * half + i
                        kk = bks[half][i]
                        fi = infls[half][i]
                        mbase = kk * 16
                        for j in range(n_vreg):
                            f = feats_v[hh, pl.ds(j * 16, 16)]
                            wv = w_v[kk, pl.ds(j * 16, 16)]
                            mexp = plsc.load_gather(
                                mod_v, [mbase + 2 * j + grp])
                            acc[j] = acc[j] + (f * fi) * (wv * mexp)
                for j in range(n_vreg):
                    outbuf_v[g2, pl.ds(j * 16, 16)] = acc[j]
                return 0

            lax.fori_loop(0, 40, m_body, 0)
            pltpu.sync_copy(outbuf_v, out_hbm.at[pl.ds(base + cc * 40, 40)])
            return 0

        lax.fori_loop(0, nm // 40, chunk_body, 0)

    return sc_kernel(q4, s_pts, s_feats, neighb_inds, mod, weights,
                     kernel_points)


def kernel(q_pts, s_pts, s_feats, neighb_inds, da_scale, W1, b1, W2, weights,
           kernel_points):
    mod = _mlp(s_feats, W1, b1, W2)
    q4 = jnp.concatenate([q_pts, da_scale[:, None]], axis=1)
    return _sc_aggregate(q4, s_pts, s_feats, neighb_inds, mod, weights,
                         kernel_points)


# speedup vs baseline: 1.3805x; 1.1539x over previous
import functools

import jax
import jax.numpy as jnp
from jax import lax
from jax.experimental import pallas as pl
from jax.experimental.pallas import tpu as pltpu
from jax.experimental.pallas import tpu_sc as plsc

_SIGMA = 2.0
_NEG_SLOPE = 0.1


def _mlp_body(x_ref, w1_ref, b1_ref, w2_ref, out_ref):
    h = jnp.dot(x_ref[...], w1_ref[...], preferred_element_type=jnp.float32)
    h = h + b1_ref[...]
    h = jnp.where(h >= 0.0, h, _NEG_SLOPE * h)
    z = jnp.dot(h, w2_ref[...], preferred_element_type=jnp.float32)
    out_ref[...] = jax.nn.sigmoid(z)


def _mlp(s_feats, W1, b1, W2):
    m, c = s_feats.shape
    cout = W2.shape[1]
    bm = 400
    assert m % bm == 0
    return pl.pallas_call(
        _mlp_body,
        grid=(m // bm,),
        in_specs=[
            pl.BlockSpec((bm, c), lambda i: (i, 0)),
            pl.BlockSpec((c, c), lambda i: (0, 0)),
            pl.BlockSpec((1, c), lambda i: (0, 0)),
            pl.BlockSpec((c, cout), lambda i: (0, 0)),
        ],
        out_specs=pl.BlockSpec((bm, cout), lambda i: (i, 0)),
        out_shape=jax.ShapeDtypeStruct((m, cout), jnp.float32),
    )(s_feats, W1, b1.reshape(1, c), W2)


def _sc_aggregate(q4, s_pts, s_feats, neighb_inds, mod, weights, kernel_points):
    m, c = s_feats.shape
    h = neighb_inds.shape[1]
    k = weights.shape[0]
    n_vreg = c // 16

    info = plsc.get_sparse_core_info()
    nw = info.num_cores * info.num_subcores
    nm = 320
    assert nw * nm >= m

    mesh = plsc.VectorSubcoreMesh(core_axis_name="c", subcore_axis_name="s")

    @functools.partial(
        pl.kernel,
        mesh=mesh,
        compiler_params=pltpu.CompilerParams(
            needs_layout_passes=False, use_tc_tiling_on_sc=False),
        out_type=jax.ShapeDtypeStruct((m, c), jnp.float32),
        scratch_types=[
            pltpu.VMEM((m * 3,), jnp.float32),
            pltpu.VMEM((nm, h), jnp.int32),
            pltpu.VMEM((nm, 4), jnp.float32),
            pltpu.VMEM((k, c), jnp.float32),
            pltpu.VMEM((16, 3), jnp.float32),
            pltpu.VMEM((40, mod.shape[1]), jnp.float32),
            pltpu.VMEM((h, c), jnp.float32),
            pltpu.VMEM((h, c), jnp.float32),
            pltpu.VMEM((40, c), jnp.float32),
            pltpu.SemaphoreType.DMA,
            pltpu.SemaphoreType.DMA,
        ],
    )
    def sc_kernel(q4_hbm, spts_hbm, sfeats_hbm, inds_hbm, mod_hbm, w_hbm,
                  kp_hbm, out_hbm, spts_v, inds_v, q4_v, w_v, kp_v, mod_v,
                  feats_a, feats_b, outbuf_v, sem_a, sem_b):
        wid = lax.axis_index("s") * info.num_cores + lax.axis_index("c")
        base = jnp.minimum(wid * nm, m - nm)

        pltpu.sync_copy(spts_hbm, spts_v)
        pltpu.sync_copy(inds_hbm.at[pl.ds(base, nm)], inds_v)
        pltpu.sync_copy(q4_hbm.at[pl.ds(base, nm)], q4_v)
        pltpu.sync_copy(w_hbm, w_v)
        pltpu.sync_copy(kp_hbm, kp_v.at[pl.ds(0, k)])

        lane = lax.iota(jnp.int32, 16)
        grp = jnp.right_shift(lane, 3)
        zeros16 = jnp.zeros((16,), jnp.int32)

        kpx = plsc.load_gather(kp_v, [lane, zeros16])
        kpy = plsc.load_gather(kp_v, [lane, zeros16 + 1])
        kpz = plsc.load_gather(kp_v, [lane, zeros16 + 2])

        def compute_point(g, g2, feats_v):
            qv = plsc.load_gather(
                q4_v, [jnp.full((16,), g, jnp.int32), lane & 3])
            qx, qy, qz, da = qv[0], qv[1], qv[2], qv[3]

            bks = []
            infls = []
            for half in range(h // 16):
                nidx3 = inds_v[g, pl.ds(16 * half, 16)] * 3
                nx = plsc.load_gather(spts_v, [nidx3]) - qx
                ny = plsc.load_gather(spts_v, [nidx3 + 1]) - qy
                nz = plsc.load_gather(spts_v, [nidx3 + 2]) - qz
                bmin = jnp.full((16,), 1e30, jnp.float32)
                bk = jnp.zeros((16,), jnp.int32)
                for kk in range(k):
                    dx = nx - da * kpx[kk]
                    dy = ny - da * kpy[kk]
                    dz = nz - da * kpz[kk]
                    sq = dx * dx + dy * dy + dz * dz
                    better = sq < bmin
                    bk = jnp.where(better, kk, bk)
                    bmin = jnp.where(better, sq, bmin)
                yi = jnp.int32(0x1FBD1DF5) + jnp.right_shift(
                    plsc.bitcast(bmin, jnp.int32), 1)
                y = plsc.bitcast(yi, jnp.float32)
                for _ in range(3):
                    y = 0.5 * (y + bmin / y)
                infl = jnp.maximum(1.0 - y * (1.0 / _SIGMA), 0.0)
                bks.append(bk)
                infls.append(infl)

            g2v = jnp.full((16,), g2, jnp.int32)
            acc = [jnp.zeros((16,), jnp.float32) for _ in range(n_vreg)]
            for half in range(h // 16):
                for i in range(16):
                    hh = 16 * half + i
                    kk = bks[half][i]
                    fi = infls[half][i]
                    mbase = kk * 16
                    for j in range(n_vreg):
                        f = feats_v[hh, pl.ds(j * 16, 16)]
                        wv = w_v[kk, pl.ds(j * 16, 16)]
                        mexp = plsc.load_gather(
                            mod_v, [g2v, mbase + 2 * j + grp])
                        acc[j] = acc[j] + (f * fi) * (wv * mexp)
            for j in range(n_vreg):
                outbuf_v[g2, pl.ds(j * 16, 16)] = acc[j]

        def gather_feats(g, buf, sem):
            pltpu.async_copy(sfeats_hbm.at[inds_v.at[g]], buf, sem)

        def wait_feats(buf, sem):
            pltpu.make_async_copy(sfeats_hbm.at[pl.ds(0, h)], buf, sem).wait()

        gather_feats(0, feats_a, sem_a)

        def chunk_body(cc, _):
            cbase = cc * 40
            pltpu.sync_copy(mod_hbm.at[pl.ds(base + cbase, 40)], mod_v)

            def pair_body(t, _inner):
                g = cbase + 2 * t
                gather_feats(g + 1, feats_b, sem_b)
                wait_feats(feats_a, sem_a)
                compute_point(g, 2 * t, feats_a)

                @pl.when(g + 2 < nm)
                def _():
                    gather_feats(g + 2, feats_a, sem_a)

                wait_feats(feats_b, sem_b)
                compute_point(g + 1, 2 * t + 1, feats_b)
                return 0

            lax.fori_loop(0, 20, pair_body, 0)
            pltpu.sync_copy(outbuf_v, out_hbm.at[pl.ds(base + cbase, 40)])
            return 0

        lax.fori_loop(0, nm // 40, chunk_body, 0)

    return sc_kernel(q4, s_pts.reshape(-1), s_feats, neighb_inds, mod,
                     weights, kernel_points)


def kernel(q_pts, s_pts, s_feats, neighb_inds, da_scale, W1, b1, W2, weights,
           kernel_points):
    mod = _mlp(s_feats, W1, b1, W2)
    q4 = jnp.concatenate([q_pts, da_scale[:, None]], axis=1)
    return _sc_aggregate(q4, s_pts, s_feats, neighb_inds, mod, weights,
                         kernel_points)


# speedup vs baseline: 1.5329x; 1.1104x over previous
import functools

import jax
import jax.numpy as jnp
from jax import lax
from jax.experimental import pallas as pl
from jax.experimental.pallas import tpu as pltpu
from jax.experimental.pallas import tpu_sc as plsc

_SIGMA = 2.0
_NEG_SLOPE = 0.1


def _mlp_body(x_ref, w1_ref, b1_ref, w2_ref, out_ref):
    h = jnp.dot(x_ref[...], w1_ref[...], preferred_element_type=jnp.float32)
    h = h + b1_ref[...]
    h = jnp.where(h >= 0.0, h, _NEG_SLOPE * h)
    z = jnp.dot(h, w2_ref[...], preferred_element_type=jnp.float32)
    out_ref[...] = jax.nn.sigmoid(z)


def _mlp(s_feats, W1, b1, W2):
    m, c = s_feats.shape
    cout = W2.shape[1]
    bm = 400
    assert m % bm == 0
    return pl.pallas_call(
        _mlp_body,
        grid=(m // bm,),
        in_specs=[
            pl.BlockSpec((bm, c), lambda i: (i, 0)),
            pl.BlockSpec((c, c), lambda i: (0, 0)),
            pl.BlockSpec((1, c), lambda i: (0, 0)),
            pl.BlockSpec((c, cout), lambda i: (0, 0)),
        ],
        out_specs=pl.BlockSpec((bm, cout), lambda i: (i, 0)),
        out_shape=jax.ShapeDtypeStruct((m, cout), jnp.float32),
    )(s_feats, W1, b1.reshape(1, c), W2)


def _sc_aggregate(q4, s_pts, s_feats, neighb_inds, mod, weights, kernel_points):
    m, c = s_feats.shape
    h = neighb_inds.shape[1]
    k = weights.shape[0]
    n_vreg = c // 16

    info = plsc.get_sparse_core_info()
    nw = info.num_cores * info.num_subcores
    nm = 320
    assert nw * nm >= m

    mesh = plsc.VectorSubcoreMesh(core_axis_name="c", subcore_axis_name="s")

    @functools.partial(
        pl.kernel,
        mesh=mesh,
        compiler_params=pltpu.CompilerParams(
            needs_layout_passes=False, use_tc_tiling_on_sc=False),
        out_type=jax.ShapeDtypeStruct((m, c), jnp.float32),
        scratch_types=[
            pltpu.VMEM((m * 3,), jnp.float32),
            pltpu.VMEM((nm, h), jnp.int32),
            pltpu.VMEM((nm, 4), jnp.float32),
            pltpu.VMEM((k, c), jnp.float32),
            pltpu.VMEM((16, 3), jnp.float32),
            pltpu.VMEM((40, mod.shape[1]), jnp.float32),
            pltpu.VMEM((h, c), jnp.float32),
            pltpu.VMEM((h, c), jnp.float32),
            pltpu.VMEM((40, c), jnp.float32),
            pltpu.SemaphoreType.DMA,
            pltpu.SemaphoreType.DMA,
        ],
    )
    def sc_kernel(q4_hbm, spts_hbm, sfeats_hbm, inds_hbm, mod_hbm, w_hbm,
                  kp_hbm, out_hbm, spts_v, inds_v, q4_v, w_v, kp_v, mod_v,
                  feats_a, feats_b, outbuf_v, sem_a, sem_b):
        wid = lax.axis_index("s") * info.num_cores + lax.axis_index("c")
        base = jnp.minimum(wid * nm, m - nm)

        pltpu.sync_copy(spts_hbm, spts_v)
        pltpu.sync_copy(inds_hbm.at[pl.ds(base, nm)], inds_v)
        pltpu.sync_copy(q4_hbm.at[pl.ds(base, nm)], q4_v)
        pltpu.sync_copy(w_hbm, w_v)
        pltpu.sync_copy(kp_hbm, kp_v.at[pl.ds(0, k)])

        lane = lax.iota(jnp.int32, 16)
        grp = jnp.right_shift(lane, 3)
        zeros16 = jnp.zeros((16,), jnp.int32)

        kpx = plsc.load_gather(kp_v, [lane, zeros16])
        kpy = plsc.load_gather(kp_v, [lane, zeros16 + 1])
        kpz = plsc.load_gather(kp_v, [lane, zeros16 + 2])

        def xlane(v, idxvec):
            return lax.gather(
                v, idxvec[:, None],
                dimension_numbers=lax.GatherDimensionNumbers(
                    offset_dims=(), collapsed_slice_dims=(0,),
                    start_index_map=(0,)),
                slice_sizes=(1,),
                mode=lax.GatherScatterMode.PROMISE_IN_BOUNDS)

        splats = [jnp.full((16,), i, jnp.int32) for i in range(16)]
        cvecs = [j * 16 + lane for j in range(n_vreg)]
        mvecs = [2 * j + grp for j in range(n_vreg)]

        def compute_point(g, g2, feats_v):
            qv = plsc.load_gather(
                q4_v, [jnp.full((16,), g, jnp.int32), lane & 3])
            qx, qy, qz, da = qv[0], qv[1], qv[2], qv[3]

            bks = []
            infls = []
            for half in range(h // 16):
                nidx3 = inds_v[g, pl.ds(16 * half, 16)] * 3
                nx = plsc.load_gather(spts_v, [nidx3]) - qx
                ny = plsc.load_gather(spts_v, [nidx3 + 1]) - qy
                nz = plsc.load_gather(spts_v, [nidx3 + 2]) - qz
                bmin = jnp.full((16,), 1e30, jnp.float32)
                bk = jnp.zeros((16,), jnp.int32)
                for kk in range(k):
                    dx = nx - da * kpx[kk]
                    dy = ny - da * kpy[kk]
                    dz = nz - da * kpz[kk]
                    sq = dx * dx + dy * dy + dz * dz
                    better = sq < bmin
                    bk = jnp.where(better, kk, bk)
                    bmin = jnp.where(better, sq, bmin)
                yi = jnp.int32(0x1FBD1DF5) + jnp.right_shift(
                    plsc.bitcast(bmin, jnp.int32), 1)
                y = plsc.bitcast(yi, jnp.float32)
                for _ in range(3):
                    y = 0.5 * (y + bmin / y)
                infl = jnp.maximum(1.0 - y * (1.0 / _SIGMA), 0.0)
                bks.append(bk)
                infls.append(infl)

            g2v = jnp.full((16,), g2, jnp.int32)
            acc = [jnp.zeros((16,), jnp.float32) for _ in range(n_vreg)]
            for half in range(h // 16):
                bk = bks[half]
                infl = infls[half]
                for i in range(16):
                    hh = 16 * half + i
                    bkb = xlane(bk, splats[i])
                    fib = xlane(infl, splats[i])
                    bkb16 = bkb * 16
                    for j in range(n_vreg):
                        f = feats_v[hh, pl.ds(j * 16, 16)]
                        wv = plsc.load_gather(w_v, [bkb, cvecs[j]])
                        mexp = plsc.load_gather(
                            mod_v, [g2v, bkb16 + mvecs[j]])
                        acc[j] = acc[j] + (f * fib) * (wv * mexp)
            for j in range(n_vreg):
                outbuf_v[g2, pl.ds(j * 16, 16)] = acc[j]

        def gather_feats(g, buf, sem):
            pltpu.async_copy(sfeats_hbm.at[inds_v.at[g]], buf, sem)

        def wait_feats(buf, sem):
            pltpu.make_async_copy(sfeats_hbm.at[pl.ds(0, h)], buf, sem).wait()

        gather_feats(0, feats_a, sem_a)

        def chunk_body(cc, _):
            cbase = cc * 40
            pltpu.sync_copy(mod_hbm.at[pl.ds(base + cbase, 40)], mod_v)

            def pair_body(t, _inner):
                g = cbase + 2 * t
                gather_feats(g + 1, feats_b, sem_b)
                wait_feats(feats_a, sem_a)
                compute_point(g, 2 * t, feats_a)

                @pl.when(g + 2 < nm)
                def _():
                    gather_feats(g + 2, feats_a, sem_a)

                wait_feats(feats_b, sem_b)
                compute_point(g + 1, 2 * t + 1, feats_b)
                return 0

            lax.fori_loop(0, 20, pair_body, 0)
            pltpu.sync_copy(outbuf_v, out_hbm.at[pl.ds(base + cbase, 40)])
            return 0

        lax.fori_loop(0, nm // 40, chunk_body, 0)

    return sc_kernel(q4, s_pts.reshape(-1), s_feats, neighb_inds, mod,
                     weights, kernel_points)


def kernel(q_pts, s_pts, s_feats, neighb_inds, da_scale, W1, b1, W2, weights,
           kernel_points):
    mod = _mlp(s_feats, W1, b1, W2)
    q4 = jnp.concatenate([q_pts, da_scale[:, None]], axis=1)
    return _sc_aggregate(q4, s_pts, s_feats, neighb_inds, mod, weights,
                         kernel_points)


# speedup vs baseline: 2.4007x; 1.5661x over previous
import functools

import jax
import jax.numpy as jnp
from jax import lax
from jax.experimental import pallas as pl
from jax.experimental.pallas import tpu as pltpu
from jax.experimental.pallas import tpu_sc as plsc

_SIGMA = 2.0
_NEG_SLOPE = 0.1


def _mlp_body(x_ref, w1_ref, b1_ref, w2_ref, out_ref):
    h = jnp.dot(x_ref[...], w1_ref[...], preferred_element_type=jnp.float32)
    h = h + b1_ref[...]
    h = jnp.where(h >= 0.0, h, _NEG_SLOPE * h)
    z = jnp.dot(h, w2_ref[...], preferred_element_type=jnp.float32)
    out_ref[...] = jax.nn.sigmoid(z)


def _mlp(s_feats, W1, b1, W2):
    m, c = s_feats.shape
    cout = W2.shape[1]
    bm = 400
    assert m % bm == 0
    return pl.pallas_call(
        _mlp_body,
        grid=(m // bm,),
        in_specs=[
            pl.BlockSpec((bm, c), lambda i: (i, 0)),
            pl.BlockSpec((c, c), lambda i: (0, 0)),
            pl.BlockSpec((1, c), lambda i: (0, 0)),
            pl.BlockSpec((c, cout), lambda i: (0, 0)),
        ],
        out_specs=pl.BlockSpec((bm, cout), lambda i: (i, 0)),
        out_shape=jax.ShapeDtypeStruct((m, cout), jnp.float32),
    )(s_feats, W1, b1.reshape(1, c), W2)


def _sc_aggregate(q4, s_pts, s_feats, neighb_inds, mod, weights, kernel_points):
    m, c = s_feats.shape
    h = neighb_inds.shape[1]
    k = weights.shape[0]
    n_vreg = c // 16

    info = plsc.get_sparse_core_info()
    nw = info.num_cores * info.num_subcores
    nm = 320
    assert nw * nm >= m

    mesh = plsc.VectorSubcoreMesh(core_axis_name="c", subcore_axis_name="s")

    @functools.partial(
        pl.kernel,
        mesh=mesh,
        compiler_params=pltpu.CompilerParams(
            needs_layout_passes=False, use_tc_tiling_on_sc=False),
        out_type=jax.ShapeDtypeStruct((m, c), jnp.float32),
        scratch_types=[
            pltpu.VMEM((m * 3,), jnp.float32),
            pltpu.VMEM((nm, h), jnp.int32),
            pltpu.VMEM((nm, 4), jnp.float32),
            pltpu.VMEM((k, c), jnp.float32),
            pltpu.VMEM((16, 3), jnp.float32),
            pltpu.VMEM((40, mod.shape[1]), jnp.float32),
            pltpu.VMEM((h, c), jnp.float32),
            pltpu.VMEM((h, c), jnp.float32),
            pltpu.VMEM((40, c), jnp.float32),
            pltpu.VMEM((k, c), jnp.float32),
            pltpu.SemaphoreType.DMA,
            pltpu.SemaphoreType.DMA,
        ],
    )
    def sc_kernel(q4_hbm, spts_hbm, sfeats_hbm, inds_hbm, mod_hbm, w_hbm,
                  kp_hbm, out_hbm, spts_v, inds_v, q4_v, w_v, kp_v, mod_v,
                  feats_a, feats_b, outbuf_v, cw_v, sem_a, sem_b):
        wid = lax.axis_index("s") * info.num_cores + lax.axis_index("c")
        base = jnp.minimum(wid * nm, m - nm)

        pltpu.sync_copy(spts_hbm, spts_v)
        pltpu.sync_copy(inds_hbm.at[pl.ds(base, nm)], inds_v)
        pltpu.sync_copy(q4_hbm.at[pl.ds(base, nm)], q4_v)
        pltpu.sync_copy(w_hbm, w_v)
        pltpu.sync_copy(kp_hbm, kp_v.at[pl.ds(0, k)])

        lane = lax.iota(jnp.int32, 16)
        grp = jnp.right_shift(lane, 3)
        zeros16 = jnp.zeros((16,), jnp.int32)

        kpx = plsc.load_gather(kp_v, [lane, zeros16])
        kpy = plsc.load_gather(kp_v, [lane, zeros16 + 1])
        kpz = plsc.load_gather(kp_v, [lane, zeros16 + 2])

        def xlane(v, idxvec):
            return lax.gather(
                v, idxvec[:, None],
                dimension_numbers=lax.GatherDimensionNumbers(
                    offset_dims=(), collapsed_slice_dims=(0,),
                    start_index_map=(0,)),
                slice_sizes=(1,),
                mode=lax.GatherScatterMode.PROMISE_IN_BOUNDS)

        splats = [jnp.full((16,), i, jnp.int32) for i in range(16)]
        cvecs = [j * 16 + lane for j in range(n_vreg)]
        mvecs = [2 * j + grp for j in range(n_vreg)]

        def compute_point(g, g2, feats_v):
            qv = plsc.load_gather(
                q4_v, [jnp.full((16,), g, jnp.int32), lane & 3])
            qx, qy, qz, da = qv[0], qv[1], qv[2], qv[3]

            bks = []
            infls = []
            for half in range(h // 16):
                nidx3 = inds_v[g, pl.ds(16 * half, 16)] * 3
                nx = plsc.load_gather(spts_v, [nidx3]) - qx
                ny = plsc.load_gather(spts_v, [nidx3 + 1]) - qy
                nz = plsc.load_gather(spts_v, [nidx3 + 2]) - qz
                bmin = jnp.full((16,), 1e30, jnp.float32)
                bk = jnp.zeros((16,), jnp.int32)
                for kk in range(k):
                    dx = nx - da * kpx[kk]
                    dy = ny - da * kpy[kk]
                    dz = nz - da * kpz[kk]
                    sq = dx * dx + dy * dy + dz * dz
                    better = sq < bmin
                    bk = jnp.where(better, kk, bk)
                    bmin = jnp.where(better, sq, bmin)
                yi = jnp.int32(0x1FBD1DF5) + jnp.right_shift(
                    plsc.bitcast(bmin, jnp.int32), 1)
                y = plsc.bitcast(yi, jnp.float32)
                for _ in range(3):
                    y = 0.5 * (y + bmin / y)
                infl = jnp.maximum(1.0 - y * (1.0 / _SIGMA), 0.0)
                bks.append(bk)
                infls.append(infl)

            for kk in range(k):
                mrow = mod_v[g2, pl.ds(kk * 16, 16)]
                for j in range(n_vreg):
                    cw_v[kk, pl.ds(j * 16, 16)] = (
                        w_v[kk, pl.ds(j * 16, 16)] * xlane(mrow, mvecs[j]))

            acc = [jnp.zeros((16,), jnp.float32) for _ in range(n_vreg)]
            for half in range(h // 16):
                bk = bks[half]
                infl = infls[half]
                for i in range(16):
                    hh = 16 * half + i
                    bkb = xlane(bk, splats[i])
                    fib = xlane(infl, splats[i])
                    for j in range(n_vreg):
                        f = feats_v[hh, pl.ds(j * 16, 16)]
                        cwv = plsc.load_gather(cw_v, [bkb, cvecs[j]])
                        acc[j] = acc[j] + (f * fib) * cwv
            for j in range(n_vreg):
                outbuf_v[g2, pl.ds(j * 16, 16)] = acc[j]

        def gather_feats(g, buf, sem):
            pltpu.async_copy(sfeats_hbm.at[inds_v.at[g]], buf, sem)

        def wait_feats(buf, sem):
            pltpu.make_async_copy(sfeats_hbm.at[pl.ds(0, h)], buf, sem).wait()

        gather_feats(0, feats_a, sem_a)

        def chunk_body(cc, _):
            cbase = cc * 40
            pltpu.sync_copy(mod_hbm.at[pl.ds(base + cbase, 40)], mod_v)

            def pair_body(t, _inner):
                g = cbase + 2 * t
                gather_feats(g + 1, feats_b, sem_b)
                wait_feats(feats_a, sem_a)
                compute_point(g, 2 * t, feats_a)

                @pl.when(g + 2 < nm)
                def _():
                    gather_feats(g + 2, feats_a, sem_a)

                wait_feats(feats_b, sem_b)
                compute_point(g + 1, 2 * t + 1, feats_b)
                return 0

            lax.fori_loop(0, 20, pair_body, 0)
            pltpu.sync_copy(outbuf_v, out_hbm.at[pl.ds(base + cbase, 40)])
            return 0

        lax.fori_loop(0, nm // 40, chunk_body, 0)

    return sc_kernel(q4, s_pts.reshape(-1), s_feats, neighb_inds, mod,
                     weights, kernel_points)


def kernel(q_pts, s_pts, s_feats, neighb_inds, da_scale, W1, b1, W2, weights,
           kernel_points):
    mod = _mlp(s_feats, W1, b1, W2)
    q4 = jnp.concatenate([q_pts, da_scale[:, None]], axis=1)
    return _sc_aggregate(q4, s_pts, s_feats, neighb_inds, mod, weights,
                         kernel_points)
